# baseline (speedup 1.0000x reference)
import functools

import jax
import jax.numpy as jnp
from jax import lax
from jax.experimental import pallas as pl
from jax.experimental.pallas import tpu as pltpu
from jax.experimental.pallas import tpu_sc as plsc

NC = 2
NS = 16
NW = NC * NS
CH = 128
IBF = 2
PAD = 128


def _sc_mesh():
    return plsc.VectorSubcoreMesh(
        core_axis_name="c", subcore_axis_name="s", num_cores=NC, num_subcores=NS
    )


def _rows_split(total):
    base = (total // NS) // 8 * 8
    last = total - (NS - 1) * base
    assert last % 8 == 0 and last > 0
    return base, last


def _sliced_copy(sid, total, mk_src, mk_dst):
    base, last = _rows_split(total)

    @pl.when(sid < NS - 1)
    def _():
        pltpu.sync_copy(mk_src(sid * base, base), mk_dst(sid * base, base))

    @pl.when(sid == NS - 1)
    def _():
        off = (NS - 1) * base
        pltpu.sync_copy(mk_src(off, last), mk_dst(off, last))


def _sc_degree(dst2d, zeros_cr, n):
    rpw = dst2d.shape[0] // NW
    cr = zeros_cr.shape[0]
    lanes = 16

    @functools.partial(
        pl.kernel,
        out_type=jax.ShapeDtypeStruct((NC, NS, cr, 128), jnp.float32),
        mesh=_sc_mesh(),
        scratch_types=[
            pltpu.VMEM((rpw, CH), jnp.int32),
            pltpu.VMEM((cr, 128), jnp.float32),
        ],
        compiler_params=pltpu.CompilerParams(needs_layout_passes=False),
    )
    def deg_kernel(dst_hbm, z_hbm, out_hbm, idx_v, cnt_v):
        cid = lax.axis_index("c")
        sid = lax.axis_index("s")
        wid = cid * NS + sid
        pltpu.sync_copy(z_hbm, cnt_v)
        pltpu.sync_copy(dst_hbm.at[pl.ds(wid * rpw, rpw)], idx_v)
        ones = jnp.full((lanes,), 1.0, jnp.float32)

        @pl.loop(0, rpw)
        def _(k):
            for j in range(CH // lanes):
                v = idx_v[k, pl.ds(j * lanes, lanes)]
                plsc.addupdate_scatter(
                    cnt_v, [v >> 7, v & 127], ones)

        pltpu.sync_copy(cnt_v, out_hbm.at[cid, sid])

    return deg_kernel(dst2d, zeros_cr)


def _sc_edge_agg(g, src2d, dst2d, zeros_nd):
    n, d = g.shape
    rpw = src2d.shape[0] // NW
    ib = rpw // IBF
    assert ib % 2 == 0 and ib % 8 == 0

    @functools.partial(
        pl.kernel,
        out_type=jax.ShapeDtypeStruct((NC, n, d), jnp.float32),
        mesh=_sc_mesh(),
        scratch_types=[
            pltpu.VMEM((ib, CH), jnp.int32),
            pltpu.VMEM((ib, CH), jnp.int32),
            pltpu.VMEM((CH, d), jnp.float32),
            pltpu.VMEM((CH, d), jnp.float32),
            pltpu.VMEM_SHARED((n + PAD, d), jnp.float32),
            pltpu.SemaphoreType.DMA,
            pltpu.SemaphoreType.DMA,
            pltpu.SemaphoreType.DMA,
            pltpu.SemaphoreType.DMA,
        ],
    )
    def agg_kernel(g_hbm, src_hbm, dst_hbm, z_hbm, out_hbm,
                   isv, idv, rows_a, rows_b, acc_sh,
                   sem_ga, sem_gb, sem_sa, sem_sb):
        cid = lax.axis_index("c")
        sid = lax.axis_index("s")
        wid = cid * NS + sid
        base_r, last_r = _rows_split(n + PAD)

        @pl.when(sid < NS - 1)
        def _():
            for off in range(0, base_r, CH):
                sz = min(CH, base_r - off)
                pltpu.sync_copy(z_hbm.at[pl.ds(0, sz)],
                                acc_sh.at[pl.ds(sid * base_r + off, sz)])

        @pl.when(sid == NS - 1)
        def _():
            o0 = (NS - 1) * base_r
            for off in range(0, last_r, CH):
                sz = min(CH, last_r - off)
                pltpu.sync_copy(z_hbm.at[pl.ds(0, sz)],
                                acc_sh.at[pl.ds(o0 + off, sz)])

        plsc.subcore_barrier()

        @pl.loop(0, IBF)
        def _(blk_i):
            base = wid * rpw + blk_i * ib
            pltpu.sync_copy(src_hbm.at[pl.ds(base, ib)], isv)
            pltpu.sync_copy(dst_hbm.at[pl.ds(base, ib)], idv)

            pltpu.async_copy(g_hbm.at[isv.at[0]], rows_a, sem_ga)

            @pl.loop(0, ib, step=2)
            def _(k):
                pltpu.make_async_copy(g_hbm.at[isv.at[k]], rows_a, sem_ga).wait()
                pltpu.async_copy(g_hbm.at[isv.at[k + 1]], rows_b, sem_gb)
                pltpu.sync_copy(rows_a, acc_sh.at[idv.at[k]], add=True)
                pltpu.make_async_copy(
                    g_hbm.at[isv.at[k + 1]], rows_b, sem_gb).wait()

                @pl.when(k + 2 < ib)
                def _():
                    pltpu.async_copy(g_hbm.at[isv.at[k + 2]], rows_a, sem_ga)

                pltpu.sync_copy(rows_b, acc_sh.at[idv.at[k + 1]], add=True)

        plsc.subcore_barrier()
        _sliced_copy(sid, n,
                     lambda o, c: acc_sh.at[pl.ds(o, c)],
                     lambda o, c: out_hbm.at[cid, pl.ds(o, c)])

    return agg_kernel(g, src2d, dst2d, zeros_nd)


def _tc_mm_scale(x, w, b, dinv, blk):
    n, d = x.shape

    def body(x_ref, w_ref, b_ref, dinv_ref, g_ref):
        h = jnp.dot(x_ref[...], w_ref[...], preferred_element_type=jnp.float32)
        g_ref[...] = (h + b_ref[...]) * dinv_ref[...]

    return pl.pallas_call(
        body,
        grid=(n // blk,),
        in_specs=[
            pl.BlockSpec((blk, d), lambda i: (i, 0)),
            pl.BlockSpec((d, d), lambda i: (0, 0)),
            pl.BlockSpec((1, d), lambda i: (0, 0)),
            pl.BlockSpec((blk, 1), lambda i: (i, 0)),
        ],
        out_specs=pl.BlockSpec((blk, d), lambda i: (i, 0)),
        out_shape=jax.ShapeDtypeStruct((n, d), jnp.float32),
    )(x, w, b, dinv)


def _tc_comb_mm(parts, g1, dinv, w2, b2, blk):
    n, d = g1.shape

    def body(pa_ref, pb_ref, g1_ref, dinv_ref, w_ref, b_ref,
             h1_ref, g2_ref):
        dinv = dinv_ref[...]
        h1 = jnp.maximum(dinv * (pa_ref[0] + pb_ref[0] + g1_ref[...]), 0.0)
        h1_ref[...] = h1
        h = jnp.dot(h1, w_ref[...], preferred_element_type=jnp.float32)
        g2_ref[...] = (h + b_ref[...]) * dinv

    return pl.pallas_call(
        body,
        grid=(n // blk,),
        in_specs=[
            pl.BlockSpec((1, blk, d), lambda i: (0, i, 0)),
            pl.BlockSpec((1, blk, d), lambda i: (1, i, 0)),
            pl.BlockSpec((blk, d), lambda i: (i, 0)),
            pl.BlockSpec((blk, 1), lambda i: (i, 0)),
            pl.BlockSpec((d, d), lambda i: (0, 0)),
            pl.BlockSpec((1, d), lambda i: (0, 0)),
        ],
        out_specs=[
            pl.BlockSpec((blk, d), lambda i: (i, 0)),
            pl.BlockSpec((blk, d), lambda i: (i, 0)),
        ],
        out_shape=[
            jax.ShapeDtypeStruct((n, d), jnp.float32),
            jax.ShapeDtypeStruct((n, d), jnp.float32),
        ],
    )(parts, parts, g1, dinv, w2, b2)


def _tc_comb_head(parts, g2, dinv, h1, wo, bo, blk):
    n, d = g2.shape
    nl = wo.shape[1]

    def body(pa_ref, pb_ref, g2_ref, dinv_ref, h1_ref, w_ref, b_ref,
             out_ref):
        h2 = jnp.maximum(dinv_ref[...] * (pa_ref[0] + pb_ref[0] + g2_ref[...]), 0.0)
        jk = jnp.maximum(h1_ref[...], h2)
        logits = jnp.dot(jk, w_ref[...], preferred_element_type=jnp.float32)
        logits = logits + b_ref[...]
        m = jnp.max(logits, axis=-1, keepdims=True)
        e = jnp.exp(logits - m)
        out_ref[...] = e / jnp.sum(e, axis=-1, keepdims=True)

    return pl.pallas_call(
        body,
        grid=(n // blk,),
        in_specs=[
            pl.BlockSpec((1, blk, d), lambda i: (0, i, 0)),
            pl.BlockSpec((1, blk, d), lambda i: (1, i, 0)),
            pl.BlockSpec((blk, d), lambda i: (i, 0)),
            pl.BlockSpec((blk, 1), lambda i: (i, 0)),
            pl.BlockSpec((blk, d), lambda i: (i, 0)),
            pl.BlockSpec((d, nl), lambda i: (0, 0)),
            pl.BlockSpec((1, nl), lambda i: (0, 0)),
        ],
        out_specs=pl.BlockSpec((blk, nl), lambda i: (i, 0)),
        out_shape=jax.ShapeDtypeStruct((n, nl), jnp.float32),
    )(parts, parts, g2, dinv, h1, wo, bo)


def kernel(x, edge_index, W1, b1, W2, b2, Wo, bo):
    n, d = x.shape
    e = edge_index.shape[1]

    src = edge_index[0].astype(jnp.int32)
    dst = edge_index[1].astype(jnp.int32)

    if e % NW:
        tail = NW - e % NW
        src = jnp.concatenate([src, jnp.zeros((tail,), jnp.int32)])
        dst = jnp.concatenate([dst, jnp.full((tail,), n, jnp.int32)])
        e += tail
    epw = e // NW
    rpw = (-(-epw // CH) + 7) // 8 * 8
    padw = rpw * CH - epw
    ar = jnp.arange(padw, dtype=jnp.int32)
    pad_src = jnp.broadcast_to(ar % n, (NW, padw))
    pad_dst = jnp.broadcast_to(n + ar % PAD, (NW, padw))
    src2d = jnp.concatenate([src.reshape(NW, epw), pad_src], axis=1).reshape(-1, CH)
    dst2d = jnp.concatenate([dst.reshape(NW, epw), pad_dst], axis=1).reshape(-1, CH)

    zeros_ch = jnp.zeros((CH, d), jnp.float32)
    cr = -(-(n + PAD) // 128)
    zeros_cr = jnp.zeros((cr, 128), jnp.float32)

    b1r = b1.reshape(1, d)
    b2r = b2.reshape(1, d)
    bor = bo.reshape(1, -1)

    blk = 1000

    degp = _sc_degree(dst2d, zeros_cr, n)
    degs = jnp.sum(degp, axis=(0, 1)).reshape(-1)[:n]
    dinv = lax.rsqrt(degs + 1.0)[:, None]

    g1 = _tc_mm_scale(x, W1, b1r, dinv, blk)
    p1 = _sc_edge_agg(g1, src2d, dst2d, zeros_ch)
    h1, g2 = _tc_comb_mm(p1, g1, dinv, W2, b2r, blk)
    p2 = _sc_edge_agg(g2, src2d, dst2d, zeros_ch)
    return _tc_comb_head(p2, g2, dinv, h1, Wo, bor, blk)


# speedup vs baseline: 1.0371x; 1.0371x over previous
import functools

import jax
import jax.numpy as jnp
from jax import lax
from jax.experimental import pallas as pl
from jax.experimental.pallas import tpu as pltpu
from jax.experimental.pallas import tpu_sc as plsc

NC = 2
NS = 16
NW = NC * NS
CH = 128
IBF = 2
PAD = 128


def _sc_mesh():
    return plsc.VectorSubcoreMesh(
        core_axis_name="c", subcore_axis_name="s", num_cores=NC, num_subcores=NS
    )


def _rows_split(total):
    base = (total // NS) // 8 * 8
    last = total - (NS - 1) * base
    assert last % 8 == 0 and last > 0
    return base, last


def _sliced_copy(sid, total, mk_src, mk_dst):
    base, last = _rows_split(total)

    @pl.when(sid < NS - 1)
    def _():
        pltpu.sync_copy(mk_src(sid * base, base), mk_dst(sid * base, base))

    @pl.when(sid == NS - 1)
    def _():
        off = (NS - 1) * base
        pltpu.sync_copy(mk_src(off, last), mk_dst(off, last))


def _sc_degree(dst2d, zeros_cr, n):
    rpw = dst2d.shape[0] // NW
    cr = zeros_cr.shape[0]
    lanes = 16

    @functools.partial(
        pl.kernel,
        out_type=jax.ShapeDtypeStruct((NC, NS, cr, 128), jnp.float32),
        mesh=_sc_mesh(),
        scratch_types=[
            pltpu.VMEM((rpw, CH), jnp.int32),
            pltpu.VMEM((cr, 128), jnp.float32),
        ],
        compiler_params=pltpu.CompilerParams(needs_layout_passes=False),
    )
    def deg_kernel(dst_hbm, z_hbm, out_hbm, idx_v, cnt_v):
        cid = lax.axis_index("c")
        sid = lax.axis_index("s")
        wid = cid * NS + sid
        pltpu.sync_copy(z_hbm, cnt_v)
        pltpu.sync_copy(dst_hbm.at[pl.ds(wid * rpw, rpw)], idx_v)
        ones = jnp.full((lanes,), 1.0, jnp.float32)

        @pl.loop(0, rpw)
        def _(k):
            for j in range(CH // lanes):
                v = idx_v[k, pl.ds(j * lanes, lanes)]
                plsc.addupdate_scatter(
                    cnt_v, [v >> 7, v & 127], ones)

        pltpu.sync_copy(cnt_v, out_hbm.at[cid, sid])

    return deg_kernel(dst2d, zeros_cr)


def _sc_edge_agg(g, src2d, dst2d, zeros_nd):
    n, d = g.shape
    rpw = src2d.shape[0] // NW
    ib = rpw // IBF
    assert ib % 2 == 0 and ib % 8 == 0

    @functools.partial(
        pl.kernel,
        out_type=jax.ShapeDtypeStruct((NC, n, d), jnp.float32),
        mesh=_sc_mesh(),
        scratch_types=[
            pltpu.VMEM((ib, CH), jnp.int32),
            pltpu.VMEM((ib, CH), jnp.int32),
            pltpu.VMEM((CH, d), jnp.float32),
            pltpu.VMEM((CH, d), jnp.float32),
            pltpu.VMEM_SHARED((n + PAD, d), jnp.float32),
            pltpu.SemaphoreType.DMA,
            pltpu.SemaphoreType.DMA,
            pltpu.SemaphoreType.DMA,
            pltpu.SemaphoreType.DMA,
        ],
    )
    def agg_kernel(g_hbm, src_hbm, dst_hbm, z_hbm, out_hbm,
                   isv, idv, rows_a, rows_b, acc_sh,
                   sem_ga, sem_gb, sem_sa, sem_sb):
        cid = lax.axis_index("c")
        sid = lax.axis_index("s")
        wid = cid * NS + sid
        _sliced_copy(sid, n + PAD,
                     lambda o, c: z_hbm.at[pl.ds(o, c)],
                     lambda o, c: acc_sh.at[pl.ds(o, c)])
        plsc.subcore_barrier()

        @pl.loop(0, IBF)
        def _(blk_i):
            base = wid * rpw + blk_i * ib
            pltpu.sync_copy(src_hbm.at[pl.ds(base, ib)], isv)
            pltpu.sync_copy(dst_hbm.at[pl.ds(base, ib)], idv)

            pltpu.async_copy(g_hbm.at[isv.at[0]], rows_a, sem_ga)

            @pl.loop(0, ib, step=2)
            def _(k):
                pltpu.make_async_copy(g_hbm.at[isv.at[k]], rows_a, sem_ga).wait()
                pltpu.async_copy(g_hbm.at[isv.at[k + 1]], rows_b, sem_gb)
                pltpu.sync_copy(rows_a, acc_sh.at[idv.at[k]], add=True)
                pltpu.make_async_copy(
                    g_hbm.at[isv.at[k + 1]], rows_b, sem_gb).wait()

                @pl.when(k + 2 < ib)
                def _():
                    pltpu.async_copy(g_hbm.at[isv.at[k + 2]], rows_a, sem_ga)

                pltpu.sync_copy(rows_b, acc_sh.at[idv.at[k + 1]], add=True)

        plsc.subcore_barrier()
        _sliced_copy(sid, n,
                     lambda o, c: acc_sh.at[pl.ds(o, c)],
                     lambda o, c: out_hbm.at[cid, pl.ds(o, c)])

    return agg_kernel(g, src2d, dst2d, zeros_nd)


def _tc_mm_scale(x, w, b, dinv, blk):
    n, d = x.shape

    def body(x_ref, w_ref, b_ref, dinv_ref, g_ref):
        h = jnp.dot(x_ref[...], w_ref[...], preferred_element_type=jnp.float32)
        g_ref[...] = (h + b_ref[...]) * dinv_ref[...]

    return pl.pallas_call(
        body,
        grid=(n // blk,),
        in_specs=[
            pl.BlockSpec((blk, d), lambda i: (i, 0)),
            pl.BlockSpec((d, d), lambda i: (0, 0)),
            pl.BlockSpec((1, d), lambda i: (0, 0)),
            pl.BlockSpec((blk, 1), lambda i: (i, 0)),
        ],
        out_specs=pl.BlockSpec((blk, d), lambda i: (i, 0)),
        out_shape=jax.ShapeDtypeStruct((n, d), jnp.float32),
    )(x, w, b, dinv)


def _tc_comb_mm(parts, g1, dinv, w2, b2, blk):
    n, d = g1.shape

    def body(pa_ref, pb_ref, g1_ref, dinv_ref, w_ref, b_ref,
             h1_ref, g2_ref):
        dinv = dinv_ref[...]
        h1 = jnp.maximum(dinv * (pa_ref[0] + pb_ref[0] + g1_ref[...]), 0.0)
        h1_ref[...] = h1
        h = jnp.dot(h1, w_ref[...], preferred_element_type=jnp.float32)
        g2_ref[...] = (h + b_ref[...]) * dinv

    return pl.pallas_call(
        body,
        grid=(n // blk,),
        in_specs=[
            pl.BlockSpec((1, blk, d), lambda i: (0, i, 0)),
            pl.BlockSpec((1, blk, d), lambda i: (1, i, 0)),
            pl.BlockSpec((blk, d), lambda i: (i, 0)),
            pl.BlockSpec((blk, 1), lambda i: (i, 0)),
            pl.BlockSpec((d, d), lambda i: (0, 0)),
            pl.BlockSpec((1, d), lambda i: (0, 0)),
        ],
        out_specs=[
            pl.BlockSpec((blk, d), lambda i: (i, 0)),
            pl.BlockSpec((blk, d), lambda i: (i, 0)),
        ],
        out_shape=[
            jax.ShapeDtypeStruct((n, d), jnp.float32),
            jax.ShapeDtypeStruct((n, d), jnp.float32),
        ],
    )(parts, parts, g1, dinv, w2, b2)


def _tc_comb_head(parts, g2, dinv, h1, wo, bo, blk):
    n, d = g2.shape
    nl = wo.shape[1]

    def body(pa_ref, pb_ref, g2_ref, dinv_ref, h1_ref, w_ref, b_ref,
             out_ref):
        h2 = jnp.maximum(dinv_ref[...] * (pa_ref[0] + pb_ref[0] + g2_ref[...]), 0.0)
        jk = jnp.maximum(h1_ref[...], h2)
        logits = jnp.dot(jk, w_ref[...], preferred_element_type=jnp.float32)
        logits = logits + b_ref[...]
        m = jnp.max(logits, axis=-1, keepdims=True)
        e = jnp.exp(logits - m)
        out_ref[...] = e / jnp.sum(e, axis=-1, keepdims=True)

    return pl.pallas_call(
        body,
        grid=(n // blk,),
        in_specs=[
            pl.BlockSpec((1, blk, d), lambda i: (0, i, 0)),
            pl.BlockSpec((1, blk, d), lambda i: (1, i, 0)),
            pl.BlockSpec((blk, d), lambda i: (i, 0)),
            pl.BlockSpec((blk, 1), lambda i: (i, 0)),
            pl.BlockSpec((blk, d), lambda i: (i, 0)),
            pl.BlockSpec((d, nl), lambda i: (0, 0)),
            pl.BlockSpec((1, nl), lambda i: (0, 0)),
        ],
        out_specs=pl.BlockSpec((blk, nl), lambda i: (i, 0)),
        out_shape=jax.ShapeDtypeStruct((n, nl), jnp.float32),
    )(parts, parts, g2, dinv, h1, wo, bo)


def kernel(x, edge_index, W1, b1, W2, b2, Wo, bo):
    n, d = x.shape
    e = edge_index.shape[1]

    src = edge_index[0].astype(jnp.int32)
    dst = edge_index[1].astype(jnp.int32)

    if e % NW:
        tail = NW - e % NW
        src = jnp.concatenate([src, jnp.zeros((tail,), jnp.int32)])
        dst = jnp.concatenate([dst, jnp.full((tail,), n, jnp.int32)])
        e += tail
    epw = e // NW
    rpw = (-(-epw // CH) + 7) // 8 * 8
    padw = rpw * CH - epw
    ar = jnp.arange(padw, dtype=jnp.int32)
    pad_src = jnp.broadcast_to(ar % n, (NW, padw))
    pad_dst = jnp.broadcast_to(n + ar % PAD, (NW, padw))
    src2d = jnp.concatenate([src.reshape(NW, epw), pad_src], axis=1).reshape(-1, CH)
    dst2d = jnp.concatenate([dst.reshape(NW, epw), pad_dst], axis=1).reshape(-1, CH)

    zeros_nd = jnp.zeros((n + PAD, d), jnp.float32)
    cr = -(-(n + PAD) // 128)
    zeros_cr = jnp.zeros((cr, 128), jnp.float32)

    b1r = b1.reshape(1, d)
    b2r = b2.reshape(1, d)
    bor = bo.reshape(1, -1)

    blk = 1000

    degp = _sc_degree(dst2d, zeros_cr, n)
    degs = jnp.sum(degp, axis=(0, 1)).reshape(-1)[:n]
    dinv = lax.rsqrt(degs + 1.0)[:, None]

    g1 = _tc_mm_scale(x, W1, b1r, dinv, blk)
    p1 = _sc_edge_agg(g1, src2d, dst2d, zeros_nd)
    h1, g2 = _tc_comb_mm(p1, g1, dinv, W2, b2r, blk)
    p2 = _sc_edge_agg(g2, src2d, dst2d, zeros_nd)
    return _tc_comb_head(p2, g2, dinv, h1, Wo, bor, blk)


# speedup vs baseline: 1.0541x; 1.0163x over previous
import functools

import jax
import jax.numpy as jnp
from jax import lax
from jax.experimental import pallas as pl
from jax.experimental.pallas import tpu as pltpu
from jax.experimental.pallas import tpu_sc as plsc

NC = 2
NS = 16
NW = NC * NS
CH = 128
IBF = 2
PAD = 128


def _sc_mesh():
    return plsc.VectorSubcoreMesh(
        core_axis_name="c", subcore_axis_name="s", num_cores=NC, num_subcores=NS
    )


def _rows_split(total):
    base = (total // NS) // 8 * 8
    last = total - (NS - 1) * base
    assert last % 8 == 0 and last > 0
    return base, last


def _sliced_copy(sid, total, mk_src, mk_dst):
    base, last = _rows_split(total)

    @pl.when(sid < NS - 1)
    def _():
        pltpu.sync_copy(mk_src(sid * base, base), mk_dst(sid * base, base))

    @pl.when(sid == NS - 1)
    def _():
        off = (NS - 1) * base
        pltpu.sync_copy(mk_src(off, last), mk_dst(off, last))


def _sc_degree(dst2d, zeros_cr, n):
    rpw = dst2d.shape[0] // NW
    cr = zeros_cr.shape[0]
    lanes = 16

    @functools.partial(
        pl.kernel,
        out_type=jax.ShapeDtypeStruct((NC, NS, cr, 128), jnp.float32),
        mesh=_sc_mesh(),
        scratch_types=[
            pltpu.VMEM((rpw, CH), jnp.int32),
            pltpu.VMEM((cr, 128), jnp.float32),
        ],
        compiler_params=pltpu.CompilerParams(needs_layout_passes=False),
    )
    def deg_kernel(dst_hbm, z_hbm, out_hbm, idx_v, cnt_v):
        cid = lax.axis_index("c")
        sid = lax.axis_index("s")
        wid = cid * NS + sid
        pltpu.sync_copy(z_hbm, cnt_v)
        pltpu.sync_copy(dst_hbm.at[pl.ds(wid * rpw, rpw)], idx_v)
        ones = jnp.full((lanes,), 1.0, jnp.float32)

        @pl.loop(0, rpw)
        def _(k):
            for j in range(CH // lanes):
                v = idx_v[k, pl.ds(j * lanes, lanes)]
                plsc.addupdate_scatter(
                    cnt_v, [v >> 7, v & 127], ones)

        pltpu.sync_copy(cnt_v, out_hbm.at[cid, sid])

    return deg_kernel(dst2d, zeros_cr)


def _sc_edge_agg(g, src2d, dst2d, zeros_nd):
    n, d = g.shape
    rpw = src2d.shape[0] // NW
    ib = rpw // IBF
    assert ib % 2 == 0 and ib % 8 == 0

    @functools.partial(
        pl.kernel,
        out_type=jax.ShapeDtypeStruct((NC, n, d), jnp.float32),
        mesh=_sc_mesh(),
        scratch_types=[
            pltpu.VMEM((ib, CH), jnp.int32),
            pltpu.VMEM((ib, CH), jnp.int32),
            pltpu.VMEM((CH, d), jnp.float32),
            pltpu.VMEM((CH, d), jnp.float32),
            pltpu.VMEM_SHARED((n + PAD, d), jnp.float32),
            pltpu.SemaphoreType.DMA,
            pltpu.SemaphoreType.DMA,
            pltpu.SemaphoreType.DMA,
            pltpu.SemaphoreType.DMA,
        ],
    )
    def agg_kernel(g_hbm, src_hbm, dst_hbm, z_hbm, out_hbm,
                   isv, idv, rows_a, rows_b, acc_sh,
                   sem_ga, sem_gb, sem_sa, sem_sb):
        cid = lax.axis_index("c")
        sid = lax.axis_index("s")
        wid = cid * NS + sid
        _sliced_copy(sid, n + PAD,
                     lambda o, c: z_hbm.at[pl.ds(o, c)],
                     lambda o, c: acc_sh.at[pl.ds(o, c)])
        plsc.subcore_barrier()

        @pl.loop(0, IBF)
        def _(blk_i):
            base = wid * rpw + blk_i * ib
            pltpu.sync_copy(src_hbm.at[pl.ds(base, ib)], isv)
            pltpu.sync_copy(dst_hbm.at[pl.ds(base, ib)], idv)

            pltpu.async_copy(g_hbm.at[isv.at[0]], rows_a, sem_ga)

            @pl.loop(0, ib, step=2)
            def _(k):
                pltpu.make_async_copy(g_hbm.at[isv.at[k]], rows_a, sem_ga).wait()
                pltpu.async_copy(g_hbm.at[isv.at[k + 1]], rows_b, sem_gb)
                pltpu.sync_copy(rows_a, acc_sh.at[idv.at[k]], add=True)
                pltpu.make_async_copy(
                    g_hbm.at[isv.at[k + 1]], rows_b, sem_gb).wait()

                @pl.when(k + 2 < ib)
                def _():
                    pltpu.async_copy(g_hbm.at[isv.at[k + 2]], rows_a, sem_ga)

                pltpu.sync_copy(rows_b, acc_sh.at[idv.at[k + 1]], add=True)

        plsc.subcore_barrier()
        _sliced_copy(sid, n,
                     lambda o, c: acc_sh.at[pl.ds(o, c)],
                     lambda o, c: out_hbm.at[cid, pl.ds(o, c)])

    return agg_kernel(g, src2d, dst2d, zeros_nd)


def _tc_mm(x, w, b, blk):
    n, d = x.shape

    def body(x_ref, w_ref, b_ref, p_ref):
        h = jnp.dot(x_ref[...], w_ref[...], preferred_element_type=jnp.float32)
        p_ref[...] = h + b_ref[...]

    return pl.pallas_call(
        body,
        grid=(n // blk,),
        in_specs=[
            pl.BlockSpec((blk, d), lambda i: (i, 0)),
            pl.BlockSpec((d, d), lambda i: (0, 0)),
            pl.BlockSpec((1, d), lambda i: (0, 0)),
        ],
        out_specs=pl.BlockSpec((blk, d), lambda i: (i, 0)),
        out_shape=jax.ShapeDtypeStruct((n, d), jnp.float32),
    )(x, w, b)


def _tc_scale(p, dinv, blk):
    n, d = p.shape

    def body(p_ref, dinv_ref, g_ref):
        g_ref[...] = p_ref[...] * dinv_ref[...]

    return pl.pallas_call(
        body,
        grid=(n // blk,),
        in_specs=[
            pl.BlockSpec((blk, d), lambda i: (i, 0)),
            pl.BlockSpec((blk, 1), lambda i: (i, 0)),
        ],
        out_specs=pl.BlockSpec((blk, d), lambda i: (i, 0)),
        out_shape=jax.ShapeDtypeStruct((n, d), jnp.float32),
    )(p, dinv)


def _tc_comb_mm(parts, g1, dinv, w2, b2, blk):
    n, d = g1.shape

    def body(pa_ref, pb_ref, g1_ref, dinv_ref, w_ref, b_ref,
             h1_ref, g2_ref):
        dinv = dinv_ref[...]
        h1 = jnp.maximum(dinv * (pa_ref[0] + pb_ref[0] + g1_ref[...]), 0.0)
        h1_ref[...] = h1
        h = jnp.dot(h1, w_ref[...], preferred_element_type=jnp.float32)
        g2_ref[...] = (h + b_ref[...]) * dinv

    return pl.pallas_call(
        body,
        grid=(n // blk,),
        in_specs=[
            pl.BlockSpec((1, blk, d), lambda i: (0, i, 0)),
            pl.BlockSpec((1, blk, d), lambda i: (1, i, 0)),
            pl.BlockSpec((blk, d), lambda i: (i, 0)),
            pl.BlockSpec((blk, 1), lambda i: (i, 0)),
            pl.BlockSpec((d, d), lambda i: (0, 0)),
            pl.BlockSpec((1, d), lambda i: (0, 0)),
        ],
        out_specs=[
            pl.BlockSpec((blk, d), lambda i: (i, 0)),
            pl.BlockSpec((blk, d), lambda i: (i, 0)),
        ],
        out_shape=[
            jax.ShapeDtypeStruct((n, d), jnp.float32),
            jax.ShapeDtypeStruct((n, d), jnp.float32),
        ],
    )(parts, parts, g1, dinv, w2, b2)


def _tc_comb_head(parts, g2, dinv, h1, wo, bo, blk):
    n, d = g2.shape
    nl = wo.shape[1]

    def body(pa_ref, pb_ref, g2_ref, dinv_ref, h1_ref, w_ref, b_ref,
             out_ref):
        h2 = jnp.maximum(dinv_ref[...] * (pa_ref[0] + pb_ref[0] + g2_ref[...]), 0.0)
        jk = jnp.maximum(h1_ref[...], h2)
        logits = jnp.dot(jk, w_ref[...], preferred_element_type=jnp.float32)
        logits = logits + b_ref[...]
        m = jnp.max(logits, axis=-1, keepdims=True)
        e = jnp.exp(logits - m)
        out_ref[...] = e / jnp.sum(e, axis=-1, keepdims=True)

    return pl.pallas_call(
        body,
        grid=(n // blk,),
        in_specs=[
            pl.BlockSpec((1, blk, d), lambda i: (0, i, 0)),
            pl.BlockSpec((1, blk, d), lambda i: (1, i, 0)),
            pl.BlockSpec((blk, d), lambda i: (i, 0)),
            pl.BlockSpec((blk, 1), lambda i: (i, 0)),
            pl.BlockSpec((blk, d), lambda i: (i, 0)),
            pl.BlockSpec((d, nl), lambda i: (0, 0)),
            pl.BlockSpec((1, nl), lambda i: (0, 0)),
        ],
        out_specs=pl.BlockSpec((blk, nl), lambda i: (i, 0)),
        out_shape=jax.ShapeDtypeStruct((n, nl), jnp.float32),
    )(parts, parts, g2, dinv, h1, wo, bo)


def kernel(x, edge_index, W1, b1, W2, b2, Wo, bo):
    n, d = x.shape
    e = edge_index.shape[1]

    src = edge_index[0].astype(jnp.int32)
    dst = edge_index[1].astype(jnp.int32)

    if e % NW:
        tail = NW - e % NW
        src = jnp.concatenate([src, jnp.zeros((tail,), jnp.int32)])
        dst = jnp.concatenate([dst, jnp.full((tail,), n, jnp.int32)])
        e += tail
    epw = e // NW
    rpw = (-(-epw // CH) + 7) // 8 * 8
    padw = rpw * CH - epw
    ar = jnp.arange(padw, dtype=jnp.int32)
    pad_src = jnp.broadcast_to(ar % n, (NW, padw))
    pad_dst = jnp.broadcast_to(n + ar % PAD, (NW, padw))
    src2d = jnp.concatenate([src.reshape(NW, epw), pad_src], axis=1).reshape(-1, CH)
    dst2d = jnp.concatenate([dst.reshape(NW, epw), pad_dst], axis=1).reshape(-1, CH)

    zeros_nd = jnp.zeros((n + PAD, d), jnp.float32)
    cr = -(-(n + PAD) // 128)
    zeros_cr = jnp.zeros((cr, 128), jnp.float32)

    b1r = b1.reshape(1, d)
    b2r = b2.reshape(1, d)
    bor = bo.reshape(1, -1)

    blk = 2000

    degp = _sc_degree(dst2d, zeros_cr, n)
    degs = jnp.sum(degp, axis=(0, 1)).reshape(-1)[:n]
    dinv = lax.rsqrt(degs + 1.0)[:, None]

    p1m = _tc_mm(x, W1, b1r, blk)
    g1 = _tc_scale(p1m, dinv, blk)
    p1 = _sc_edge_agg(g1, src2d, dst2d, zeros_nd)
    h1, g2 = _tc_comb_mm(p1, g1, dinv, W2, b2r, blk)
    p2 = _sc_edge_agg(g2, src2d, dst2d, zeros_nd)
    return _tc_comb_head(p2, g2, dinv, h1, Wo, bor, blk)
